# baseline (speedup 1.0000x reference)
import jax
import jax.numpy as jnp
from jax import lax
from jax.experimental import pallas as pl
from jax.experimental.pallas import tpu as pltpu
from jax.experimental.pallas import tpu_sc as plsc

N = 10000
E = 320000
D = 128
H = 8
C = 16
EPS = 1e-5

NC = 2
NS = 16
NW = NC * NS
L = 16

S = 128
ETOT = E + N
IBLK = 4
NSUB = 84
assert NSUB * NW * S >= ETOT and NSUB % IBLK == 0 and IBLK % 2 == 0
CSZ = NSUB * S
EPAD = NW * CSZ
NPAD = -(-N // (NS * 8)) * NS * 8
RPT = NPAD // NS


def _vgather(v, idx):
    return lax.gather(
        v, idx[:, None],
        lax.GatherDimensionNumbers(offset_dims=(), collapsed_slice_dims=(0,),
                                   start_index_map=(0,)),
        (1,), mode=lax.GatherScatterMode.PROMISE_IN_BOUNDS)


def _edge_body(src_hbm, dst_hbm, h_hbm, combs_hbm, combd_hbm, zacc_hbm,
               zden_hbm, acc_out, den_out,
               acc_sh, den_sh, isrc_v, idst_v, srows, drows, wden,
               rows0, rows1, sem_h0, sem_h1, sem_a, sem_b, sem_w, sem_s):
    cid = lax.axis_index("c")
    sid = lax.axis_index("s")
    wid = sid * NC + cid

    pltpu.sync_copy(zacc_hbm.at[pl.ds(sid * RPT, RPT)],
                    acc_sh.at[pl.ds(sid * RPT, RPT)])
    pltpu.sync_copy(zden_hbm.at[pl.ds(sid * RPT, RPT)],
                    den_sh.at[pl.ds(sid * RPT, RPT)])
    plsc.subcore_barrier()

    iota_f = lax.iota(jnp.int32, L).astype(jnp.float32)
    penal = jnp.minimum(float(H) - 0.5 - iota_f, 0.0) * 1e30

    def alpha_w(isrc, idst):
        cp_s = pltpu.async_copy(combs_hbm.at[isrc], srows, sem_a)
        cp_d = pltpu.async_copy(combd_hbm.at[idst], drows, sem_b)
        cp_s.wait()
        cp_d.wait()

        @plsc.parallel_loop(0, S, unroll=8)
        def wstage(e):
            ev = srows[e, :] + drows[e, :]
            ev = jnp.maximum(ev, 0.2 * ev)
            wden[e, :] = jnp.exp(ev + penal)

    def scale(rows):
        @plsc.parallel_loop(0, S, unroll=4)
        def _s(e):
            w = wden[e, :]
            for h in range(H):
                sl = pl.ds(h * C, C)
                rows[e, sl] = rows[e, sl] * _vgather(
                    w, jnp.full((L,), h, jnp.int32))

    def pair(t, carry):
        j0 = t * 2
        j1 = j0 + 1
        is0, id0 = isrc_v.at[j0], idst_v.at[j0]
        is1, id1 = isrc_v.at[j1], idst_v.at[j1]
        cp_h0 = pltpu.async_copy(h_hbm.at[is0], rows0, sem_h0)
        cp_h1 = pltpu.async_copy(h_hbm.at[is1], rows1, sem_h1)
        alpha_w(is0, id0)
        cp_w0 = pltpu.async_copy(wden, den_sh.at[id0], sem_w, add=True)
        cp_h0.wait()
        scale(rows0)
        cp_s0 = pltpu.async_copy(rows0, acc_sh.at[id0], sem_s, add=True)
        cp_w0.wait()
        alpha_w(is1, id1)
        cp_w1 = pltpu.async_copy(wden, den_sh.at[id1], sem_w, add=True)
        cp_h1.wait()
        scale(rows1)
        cp_s0.wait()
        cp_s1 = pltpu.async_copy(rows1, acc_sh.at[id1], sem_s, add=True)
        cp_w1.wait()
        cp_s1.wait()
        return carry

    def blk(bi, carry):
        pltpu.sync_copy(src_hbm.at[wid, pl.ds(bi * IBLK, IBLK)], isrc_v)
        pltpu.sync_copy(dst_hbm.at[wid, pl.ds(bi * IBLK, IBLK)], idst_v)
        lax.fori_loop(0, IBLK // 2, pair, 0)
        return carry

    lax.fori_loop(0, NSUB // IBLK, blk, 0)
    plsc.subcore_barrier()

    pltpu.sync_copy(acc_sh.at[pl.ds(sid * RPT, RPT)],
                    acc_out.at[cid, pl.ds(sid * RPT, RPT)])
    pltpu.sync_copy(den_sh.at[pl.ds(sid * RPT, RPT)],
                    den_out.at[cid, pl.ds(sid * RPT, RPT)])


def _edge(src3, dst3, h, combs, combd, zacc, zden):
    fn = pl.kernel(
        _edge_body,
        out_type=[jax.ShapeDtypeStruct((NC, NPAD, D), jnp.float32),
                  jax.ShapeDtypeStruct((NC, NPAD, L), jnp.float32)],
        mesh=plsc.VectorSubcoreMesh(core_axis_name="c", subcore_axis_name="s",
                                    num_cores=NC, num_subcores=NS),
        scratch_types=[
            pltpu.VMEM_SHARED((NPAD, D), jnp.float32),
            pltpu.VMEM_SHARED((NPAD, L), jnp.float32),
            pltpu.VMEM((IBLK, S), jnp.int32),
            pltpu.VMEM((IBLK, S), jnp.int32),
            pltpu.VMEM((S, L), jnp.float32),
            pltpu.VMEM((S, L), jnp.float32),
            pltpu.VMEM((S, L), jnp.float32),
            pltpu.VMEM((S, D), jnp.float32),
            pltpu.VMEM((S, D), jnp.float32),
            pltpu.SemaphoreType.DMA,
            pltpu.SemaphoreType.DMA,
            pltpu.SemaphoreType.DMA,
            pltpu.SemaphoreType.DMA,
            pltpu.SemaphoreType.DMA,
            pltpu.SemaphoreType.DMA,
        ],
        compiler_params=pltpu.CompilerParams(use_tc_tiling_on_sc=False),
    )
    return fn(src3, dst3, h, combs, combd, zacc, zden)


def _comb_mat(a_flat):
    r = lax.broadcasted_iota(jnp.int32, (D, H), 0) // C
    c = lax.broadcasted_iota(jnp.int32, (D, H), 1)
    m = (r == c).astype(jnp.float32)
    return jnp.concatenate([a_flat[:, None] * m, jnp.zeros((D, H))], axis=1)


def _expand_mask():
    r = lax.broadcasted_iota(jnp.int32, (H, D), 0)
    c = lax.broadcasted_iota(jnp.int32, (H, D), 1) // C
    return (r == c).astype(jnp.float32)


def _init_body(x_ref, wi_ref, w0_ref, as_ref, ad_ref, h_ref, cs_ref, cd_ref):
    h0 = jnp.dot(x_ref[...], wi_ref[...], preferred_element_type=jnp.float32)
    h = jnp.dot(h0, w0_ref[...], preferred_element_type=jnp.float32)
    h_ref[...] = h
    cs_ref[...] = jnp.dot(h, _comb_mat(as_ref[...]),
                          preferred_element_type=jnp.float32)
    cd_ref[...] = jnp.dot(h, _comb_mat(ad_ref[...]),
                          preferred_element_type=jnp.float32)


def _combine(acc_ref, den_ref, b_ref):
    acc = acc_ref[0] + acc_ref[1]
    den = den_ref[0] + den_ref[1]
    dchan = jnp.dot(den[:N, :H], _expand_mask(),
                    preferred_element_type=jnp.float32)
    return acc[:N] / (dchan + 1e-16) + b_ref[...]


def _mid_body(acc_ref, den_ref, b_ref, g_ref, be_ref, w_ref, as_ref, ad_ref,
              h_ref, cs_ref, cd_ref):
    y = _combine(acc_ref, den_ref, b_ref)
    mu = jnp.mean(y, axis=0)
    var = jnp.mean((y - mu) ** 2, axis=0)
    xn = (y - mu) / jnp.sqrt(var + EPS) * g_ref[...] + be_ref[...]
    h = jnp.dot(xn, w_ref[...], preferred_element_type=jnp.float32)
    h_ref[...] = h
    cs_ref[...] = jnp.dot(h, _comb_mat(as_ref[...]),
                          preferred_element_type=jnp.float32)
    cd_ref[...] = jnp.dot(h, _comb_mat(ad_ref[...]),
                          preferred_element_type=jnp.float32)


def _final_body(acc_ref, den_ref, b_ref, o_ref):
    o_ref[...] = _combine(acc_ref, den_ref, b_ref)


_HC_OUT = [jax.ShapeDtypeStruct((N, D), jnp.float32),
           jax.ShapeDtypeStruct((N, 2 * H), jnp.float32),
           jax.ShapeDtypeStruct((N, 2 * H), jnp.float32)]


def kernel(x, edge_index, W_init, W0, a_src0, a_dst0, b0,
           W1, a_src1, a_dst1, b1, W2, a_src2, a_dst2, b2,
           gamma0, beta0, gamma1, beta1):
    loop = jnp.arange(N, dtype=jnp.int32)
    pad = EPAD - ETOT
    src = jnp.concatenate([edge_index[0].astype(jnp.int32), loop,
                           jnp.zeros((pad,), jnp.int32)]).reshape(NW, NSUB, S)
    pad_dst = N + jnp.arange(pad, dtype=jnp.int32) % (NPAD - N)
    dst = jnp.concatenate([edge_index[1].astype(jnp.int32), loop,
                           pad_dst]).reshape(NW, NSUB, S)
    zacc = jnp.zeros((NPAD, D), jnp.float32)
    zden = jnp.zeros((NPAD, L), jnp.float32)

    tc_init = pl.pallas_call(_init_body, out_shape=_HC_OUT)
    tc_mid = pl.pallas_call(_mid_body, out_shape=_HC_OUT)
    tc_final = pl.pallas_call(
        _final_body, out_shape=jax.ShapeDtypeStruct((N, D), jnp.float32))

    h, cs, cd = tc_init(x, W_init, W0, a_src0.reshape(D), a_dst0.reshape(D))
    acc, den = _edge(src, dst, h, cs, cd, zacc, zden)
    h, cs, cd = tc_mid(acc, den, b0, gamma0, beta0, W1,
                       a_src1.reshape(D), a_dst1.reshape(D))
    acc, den = _edge(src, dst, h, cs, cd, zacc, zden)
    h, cs, cd = tc_mid(acc, den, b1, gamma1, beta1, W2,
                       a_src2.reshape(D), a_dst2.reshape(D))
    acc, den = _edge(src, dst, h, cs, cd, zacc, zden)
    return tc_final(acc, den, b2)


# speedup vs baseline: 2.3268x; 2.3268x over previous
import jax
import jax.numpy as jnp
from jax import lax
from jax.experimental import pallas as pl
from jax.experimental.pallas import tpu as pltpu
from jax.experimental.pallas import tpu_sc as plsc

N = 10000
E = 320000
D = 128
H = 8
C = 16
EPS = 1e-5

NC = 2
NS = 16
NW = NC * NS
L = 16

S = 128
ETOT = E + N
IBLK = 9
NSUB = 81
assert NSUB * NW * S >= ETOT and NSUB % IBLK == 0
CSZ = NSUB * S
EPAD = NW * CSZ
NPAD = -(-N // (NS * 8)) * NS * 8
RPT = NPAD // NS


def _vgather(v, idx):
    return lax.gather(
        v, idx[:, None],
        lax.GatherDimensionNumbers(offset_dims=(), collapsed_slice_dims=(0,),
                                   start_index_map=(0,)),
        (1,), mode=lax.GatherScatterMode.PROMISE_IN_BOUNDS)


def _edge_body(src_hbm, dst_hbm, h_hbm, combs_hbm, combd_hbm, zacc_hbm,
               zden_hbm, acc_out, den_out,
               acc_sh, den_sh, isrc_v, idst_v, srows, drows, wden,
               rows0, sem_h0, sem_a, sem_b, sem_w, sem_s):
    cid = lax.axis_index("c")
    sid = lax.axis_index("s")
    wid = sid * NC + cid

    pltpu.sync_copy(zacc_hbm.at[pl.ds(sid * RPT, RPT)],
                    acc_sh.at[pl.ds(sid * RPT, RPT)])
    pltpu.sync_copy(zden_hbm.at[pl.ds(sid * RPT, RPT)],
                    den_sh.at[pl.ds(sid * RPT, RPT)])
    plsc.subcore_barrier()

    iota_f = lax.iota(jnp.int32, L).astype(jnp.float32)
    penal = jnp.minimum(float(H) - 0.5 - iota_f, 0.0) * 1e30

    def alpha_w(isrc, idst):
        cp_s = pltpu.async_copy(combs_hbm.at[isrc], srows, sem_a)
        cp_d = pltpu.async_copy(combd_hbm.at[idst], drows, sem_b)
        cp_s.wait()
        cp_d.wait()

        @plsc.parallel_loop(0, S, unroll=8)
        def wstage(e):
            ev = srows[e, :] + drows[e, :]
            ev = jnp.maximum(ev, 0.2 * ev)
            wden[e, :] = jnp.exp(ev + penal)

    def scale(rows):
        @plsc.parallel_loop(0, S, unroll=4)
        def _s(e):
            w = wden[e, :]
            for h in range(H):
                sl = pl.ds(h * C, C)
                rows[e, sl] = rows[e, sl] * _vgather(
                    w, jnp.full((L,), h, jnp.int32))

    def sub(j, carry):
        isrc, idst = isrc_v.at[j], idst_v.at[j]
        cp_h = pltpu.async_copy(h_hbm.at[isrc], rows0, sem_h0)
        alpha_w(isrc, idst)
        cp_w = pltpu.async_copy(wden, den_sh.at[idst], sem_w, add=True)
        cp_h.wait()
        scale(rows0)
        cp_s = pltpu.async_copy(rows0, acc_sh.at[idst], sem_s, add=True)
        cp_w.wait()
        cp_s.wait()
        return carry

    def blk(bi, carry):
        pltpu.sync_copy(src_hbm.at[wid, pl.ds(bi * IBLK, IBLK)], isrc_v)
        pltpu.sync_copy(dst_hbm.at[wid, pl.ds(bi * IBLK, IBLK)], idst_v)
        lax.fori_loop(0, IBLK, sub, 0)
        return carry

    lax.fori_loop(0, NSUB // IBLK, blk, 0)
    plsc.subcore_barrier()

    pltpu.sync_copy(acc_sh.at[pl.ds(sid * RPT, RPT)],
                    acc_out.at[cid, pl.ds(sid * RPT, RPT)])
    pltpu.sync_copy(den_sh.at[pl.ds(sid * RPT, RPT)],
                    den_out.at[cid, pl.ds(sid * RPT, RPT)])


def _edge(src3, dst3, h, combs, combd, zacc, zden):
    fn = pl.kernel(
        _edge_body,
        out_type=[jax.ShapeDtypeStruct((NC, NPAD, D), jnp.float32),
                  jax.ShapeDtypeStruct((NC, NPAD, L), jnp.float32)],
        mesh=plsc.VectorSubcoreMesh(core_axis_name="c", subcore_axis_name="s",
                                    num_cores=NC, num_subcores=NS),
        scratch_types=[
            pltpu.VMEM_SHARED((NPAD, D), jnp.float32),
            pltpu.VMEM_SHARED((NPAD, L), jnp.float32),
            pltpu.VMEM((IBLK, S), jnp.int32),
            pltpu.VMEM((IBLK, S), jnp.int32),
            pltpu.VMEM((S, L), jnp.float32),
            pltpu.VMEM((S, L), jnp.float32),
            pltpu.VMEM((S, L), jnp.float32),
            pltpu.VMEM((S, D), jnp.float32),
            pltpu.SemaphoreType.DMA,
            pltpu.SemaphoreType.DMA,
            pltpu.SemaphoreType.DMA,
            pltpu.SemaphoreType.DMA,
            pltpu.SemaphoreType.DMA,
        ],
        compiler_params=pltpu.CompilerParams(use_tc_tiling_on_sc=False),
    )
    return fn(src3, dst3, h, combs, combd, zacc, zden)


def _comb_mat(a_flat):
    r = lax.broadcasted_iota(jnp.int32, (D, H), 0) // C
    c = lax.broadcasted_iota(jnp.int32, (D, H), 1)
    m = (r == c).astype(jnp.float32)
    return jnp.concatenate([a_flat[:, None] * m, jnp.zeros((D, H))], axis=1)


def _expand_mask():
    r = lax.broadcasted_iota(jnp.int32, (H, D), 0)
    c = lax.broadcasted_iota(jnp.int32, (H, D), 1) // C
    return (r == c).astype(jnp.float32)


def _init_body(x_ref, wi_ref, w0_ref, as_ref, ad_ref, h_ref, cs_ref, cd_ref):
    h0 = jnp.dot(x_ref[...], wi_ref[...], preferred_element_type=jnp.float32)
    h = jnp.dot(h0, w0_ref[...], preferred_element_type=jnp.float32)
    h_ref[...] = h
    cs_ref[...] = jnp.dot(h, _comb_mat(as_ref[...]),
                          preferred_element_type=jnp.float32)
    cd_ref[...] = jnp.dot(h, _comb_mat(ad_ref[...]),
                          preferred_element_type=jnp.float32)


def _combine(acc_ref, den_ref, b_ref):
    acc = acc_ref[0] + acc_ref[1]
    den = den_ref[0] + den_ref[1]
    dchan = jnp.dot(den[:N, :H], _expand_mask(),
                    preferred_element_type=jnp.float32)
    return acc[:N] / (dchan + 1e-16) + b_ref[...]


def _mid_body(acc_ref, den_ref, b_ref, g_ref, be_ref, w_ref, as_ref, ad_ref,
              h_ref, cs_ref, cd_ref):
    y = _combine(acc_ref, den_ref, b_ref)
    mu = jnp.mean(y, axis=0)
    var = jnp.mean((y - mu) ** 2, axis=0)
    xn = (y - mu) / jnp.sqrt(var + EPS) * g_ref[...] + be_ref[...]
    h = jnp.dot(xn, w_ref[...], preferred_element_type=jnp.float32)
    h_ref[...] = h
    cs_ref[...] = jnp.dot(h, _comb_mat(as_ref[...]),
                          preferred_element_type=jnp.float32)
    cd_ref[...] = jnp.dot(h, _comb_mat(ad_ref[...]),
                          preferred_element_type=jnp.float32)


def _final_body(acc_ref, den_ref, b_ref, o_ref):
    o_ref[...] = _combine(acc_ref, den_ref, b_ref)


_HC_OUT = [jax.ShapeDtypeStruct((N, D), jnp.float32),
           jax.ShapeDtypeStruct((N, 2 * H), jnp.float32),
           jax.ShapeDtypeStruct((N, 2 * H), jnp.float32)]


def kernel(x, edge_index, W_init, W0, a_src0, a_dst0, b0,
           W1, a_src1, a_dst1, b1, W2, a_src2, a_dst2, b2,
           gamma0, beta0, gamma1, beta1):
    loop = jnp.arange(N, dtype=jnp.int32)
    pad = EPAD - ETOT
    src = jnp.concatenate([edge_index[0].astype(jnp.int32), loop,
                           jnp.zeros((pad,), jnp.int32)]).reshape(NW, NSUB, S)
    pad_dst = N + jnp.arange(pad, dtype=jnp.int32) % (NPAD - N)
    dst = jnp.concatenate([edge_index[1].astype(jnp.int32), loop,
                           pad_dst]).reshape(NW, NSUB, S)
    zacc = jnp.zeros((NPAD, D), jnp.float32)
    zden = jnp.zeros((NPAD, L), jnp.float32)

    tc_init = pl.pallas_call(_init_body, out_shape=_HC_OUT)
    tc_mid = pl.pallas_call(_mid_body, out_shape=_HC_OUT)
    tc_final = pl.pallas_call(
        _final_body, out_shape=jax.ShapeDtypeStruct((N, D), jnp.float32))

    h, cs, cd = tc_init(x, W_init, W0, a_src0.reshape(D), a_dst0.reshape(D))
    acc, den = _edge(src, dst, h, cs, cd, zacc, zden)
    h, cs, cd = tc_mid(acc, den, b0, gamma0, beta0, W1,
                       a_src1.reshape(D), a_dst1.reshape(D))
    acc, den = _edge(src, dst, h, cs, cd, zacc, zden)
    h, cs, cd = tc_mid(acc, den, b1, gamma1, beta1, W2,
                       a_src2.reshape(D), a_dst2.reshape(D))
    acc, den = _edge(src, dst, h, cs, cd, zacc, zden)
    return tc_final(acc, den, b2)
